# baseline (speedup 1.0000x reference)
import functools

import jax
import jax.numpy as jnp
from jax import lax
from jax.experimental import pallas as pl
from jax.experimental.pallas import tpu as pltpu
from jax.experimental.pallas import tpu_sc as plsc

N = 10000
E = 640000
D = 128
G = 64
NC = 2

NCORE = 2
NSUB = 16
NW = NCORE * NSUB
K = 80
NCHUNK = 256
EPW = NCHUNK * K
EP = NW * EPW
NP = 10240
RPS = NP // NSUB

BR = 400
NB = N // BR
BRM = 512
NBM = NP // BRM


def _edge_agg(h, src, dst, zeros):
    mesh = plsc.VectorSubcoreMesh(core_axis_name="c", subcore_axis_name="s")

    @functools.partial(
        pl.kernel,
        out_type=jax.ShapeDtypeStruct((NCORE * NP, D), jnp.float32),
        mesh=mesh,
        scratch_types=(
            [pltpu.VMEM((K,), jnp.int32)] * 8
            + [pltpu.VMEM((K, D), jnp.float32)] * 4
            + [pltpu.VMEM_SHARED((NP, D), jnp.float32)]
            + [pltpu.SemaphoreType.DMA] * 12
        ),
    )
    def k(h_hbm, src_hbm, dst_hbm, z_hbm, out_hbm, *bufs):
        sidx = bufs[0:4]
        didx = bufs[4:8]
        rows = bufs[8:12]
        acc = bufs[12]
        gsem = bufs[13:17]
        ssem = bufs[17:21]
        isem = bufs[21:25]

        cid = lax.axis_index("c")
        sid = lax.axis_index("s")
        wid = cid * NSUB + sid

        pltpu.sync_copy(z_hbm.at[pl.ds(sid * RPS, RPS)],
                        acc.at[pl.ds(sid * RPS, RPS)])
        plsc.subcore_barrier()

        base = wid * EPW

        def idx_start(c, p):
            off = base + c * K
            pltpu.async_copy(src_hbm.at[pl.ds(off, K)], sidx[p], isem[p])
            pltpu.async_copy(dst_hbm.at[pl.ds(off, K)], didx[p], isem[p])

        def idx_wait(p):
            pltpu.make_async_copy(src_hbm.at[pl.ds(0, K)], sidx[p],
                                  isem[p]).wait()
            pltpu.make_async_copy(dst_hbm.at[pl.ds(0, K)], didx[p],
                                  isem[p]).wait()

        def gather_start(p):
            pltpu.async_copy(h_hbm.at[sidx[p]], rows[p], gsem[p])

        def gather_wait(p):
            pltpu.make_async_copy(z_hbm.at[pl.ds(0, K)], rows[p],
                                  gsem[p]).wait()

        def scatter_start(p):
            pltpu.async_copy(rows[p], acc.at[didx[p]], ssem[p], add=True)

        def scatter_wait(p):
            pltpu.make_async_copy(rows[p], acc.at[didx[p]], ssem[p]).wait()

        idx_start(0, 0)
        idx_wait(0)
        gather_start(0)
        idx_start(1, 1)

        def body(g, _):
            for u in range(4):
                c = 4 * g + u
                gather_wait(u)
                scatter_start(u)

                @pl.when(c >= 2)
                def _():
                    scatter_wait((u + 2) % 4)

                @pl.when(c + 2 < NCHUNK)
                def _():
                    idx_start(c + 2, (u + 2) % 4)

                @pl.when(c + 1 < NCHUNK)
                def _():
                    idx_wait((u + 1) % 4)
                    gather_start((u + 1) % 4)
            return 0

        lax.fori_loop(0, NCHUNK // 4, body, 0)
        scatter_wait((NCHUNK - 2) % 4)
        scatter_wait((NCHUNK - 1) % 4)
        plsc.subcore_barrier()
        pltpu.sync_copy(acc.at[pl.ds(sid * RPS, RPS)],
                        out_hbm.at[pl.ds(cid * NP + sid * RPS, RPS)])

    return k(h, src, dst, zeros)


def _mlp_layer(x, parts, Wa, ba, Wb, bb):
    def body(x_ref, p0_ref, p1_ref, wa_ref, ba_ref, wb_ref, bb_ref, o_ref):
        m = x_ref[...] + p0_ref[...] + p1_ref[...]
        t = jnp.maximum(
            jnp.dot(m, wa_ref[...], preferred_element_type=jnp.float32)
            + ba_ref[...], 0.0)
        o_ref[...] = jnp.maximum(
            jnp.dot(t, wb_ref[...], preferred_element_type=jnp.float32)
            + bb_ref[...], 0.0)

    return pl.pallas_call(
        body,
        grid=(NBM,),
        in_specs=[
            pl.BlockSpec((BRM, D), lambda i: (i, 0)),
            pl.BlockSpec((BRM, D), lambda i: (i, 0)),
            pl.BlockSpec((BRM, D), lambda i: (i + NBM, 0)),
            pl.BlockSpec((D, D), lambda i: (0, 0)),
            pl.BlockSpec((1, D), lambda i: (0, 0)),
            pl.BlockSpec((D, D), lambda i: (0, 0)),
            pl.BlockSpec((1, D), lambda i: (0, 0)),
        ],
        out_specs=pl.BlockSpec((BRM, D), lambda i: (i, 0)),
        out_shape=jax.ShapeDtypeStruct((NP, D), jnp.float32),
    )(x, parts, parts, Wa, ba.reshape(1, D), Wb, bb.reshape(1, D))


def _head(h1, h2, h3, batch3, Wjk, bjk, Wc1, bc1, g1, bt1, Wc2, bc2):
    inv = float((1.0 + 1e-5) ** -0.5)

    def body(h1_ref, h2_ref, h3_ref, b_ref, wjk_ref, bjk_ref, wc1_ref,
             bc1_ref, g1_ref, bt1_ref, wc2_ref, bc2_ref, o_ref, pooled):
        i = pl.program_id(0)

        @pl.when(i == 0)
        def _():
            pooled[...] = jnp.zeros_like(pooled)

        hcat = jnp.concatenate([h1_ref[...], h2_ref[...], h3_ref[...]],
                               axis=-1)
        xo = jnp.dot(hcat, wjk_ref[...],
                     preferred_element_type=jnp.float32) + bjk_ref[...]
        seg = b_ref[0, 0, :]
        onehot = (jax.lax.broadcasted_iota(jnp.int32, (G, BR), 0)
                  == seg[None, :]).astype(jnp.float32)
        pooled[...] += jnp.dot(onehot, xo, preferred_element_type=jnp.float32)

        @pl.when(i == NB - 1)
        def _():
            c = jnp.dot(pooled[...], wc1_ref[...],
                        preferred_element_type=jnp.float32) + bc1_ref[...]
            c = c * inv * g1_ref[...] + bt1_ref[...]
            c = jnp.maximum(c, 0.0)
            o_ref[...] = jnp.dot(c, wc2_ref[...],
                                 preferred_element_type=jnp.float32) + bc2_ref[...]

    return pl.pallas_call(
        body,
        grid=(NB,),
        in_specs=[
            pl.BlockSpec((BR, D), lambda i: (i, 0)),
            pl.BlockSpec((BR, D), lambda i: (i, 0)),
            pl.BlockSpec((BR, D), lambda i: (i, 0)),
            pl.BlockSpec((1, 1, BR), lambda i: (i, 0, 0)),
            pl.BlockSpec((3 * D, D), lambda i: (0, 0)),
            pl.BlockSpec((1, D), lambda i: (0, 0)),
            pl.BlockSpec((D, 2 * D), lambda i: (0, 0)),
            pl.BlockSpec((1, 2 * D), lambda i: (0, 0)),
            pl.BlockSpec((1, 2 * D), lambda i: (0, 0)),
            pl.BlockSpec((1, 2 * D), lambda i: (0, 0)),
            pl.BlockSpec((2 * D, NC), lambda i: (0, 0)),
            pl.BlockSpec((1, NC), lambda i: (0, 0)),
        ],
        out_specs=pl.BlockSpec((G, NC), lambda i: (0, 0)),
        out_shape=jax.ShapeDtypeStruct((G, NC), jnp.float32),
        scratch_shapes=[pltpu.VMEM((G, D), jnp.float32)],
    )(h1, h2, h3, batch3, Wjk, bjk.reshape(1, D), Wc1, bc1.reshape(1, 2 * D),
      g1.reshape(1, 2 * D), bt1.reshape(1, 2 * D), Wc2, bc2.reshape(1, NC))


def kernel(x, edge_index, batch, W1a, b1a, W1b, b1b, W2a, b2a, W2b, b2b,
           W3a, b3a, W3b, b3b, Wjk, bjk, Wc1, bc1, g1, bt1, Wc2, bc2):
    src = jnp.pad(edge_index[0], (0, EP - E))
    dst = jnp.pad(edge_index[1], (0, EP - E), constant_values=N)
    zeros = jnp.zeros((NP, D), jnp.float32)
    batch3 = batch.reshape(NB, 1, BR)
    xp = jnp.pad(x, ((0, NP - N), (0, 0)))

    p1 = _edge_agg(xp, src, dst, zeros)
    h1 = _mlp_layer(xp, p1, W1a, b1a, W1b, b1b)
    p2 = _edge_agg(h1, src, dst, zeros)
    h2 = _mlp_layer(h1, p2, W2a, b2a, W2b, b2b)
    p3 = _edge_agg(h2, src, dst, zeros)
    h3 = _mlp_layer(h2, p3, W3a, b3a, W3b, b3b)

    return _head(h1, h2, h3, batch3, Wjk, bjk, Wc1, bc1, g1, bt1, Wc2, bc2)


# speedup vs baseline: 1.0624x; 1.0624x over previous
import functools

import jax
import jax.numpy as jnp
from jax import lax
from jax.experimental import pallas as pl
from jax.experimental.pallas import tpu as pltpu
from jax.experimental.pallas import tpu_sc as plsc

N = 10000
E = 640000
D = 128
G = 64
NC = 2

NCORE = 2
NSUB = 16
NW = NCORE * NSUB
K = 80
NCHUNK = 256
EPW = NCHUNK * K
EP = NW * EPW
NP = 10240
RPS = NP // NSUB

BR = 400
NB = N // BR
BRM = 512
NBM = NP // BRM


def _edge_agg(h, src, dst, zeros):
    mesh = plsc.VectorSubcoreMesh(core_axis_name="c", subcore_axis_name="s")

    @functools.partial(
        pl.kernel,
        out_type=jax.ShapeDtypeStruct((NCORE * NP, D), jnp.float32),
        mesh=mesh,
        scratch_types=(
            [pltpu.VMEM((K,), jnp.int32)] * 8
            + [pltpu.VMEM((K, D), jnp.float32)] * 4
            + [pltpu.VMEM_SHARED((NP, D), jnp.float32)]
            + [pltpu.SemaphoreType.DMA] * 12
        ),
    )
    def k(h_hbm, src_hbm, dst_hbm, z_hbm, out_hbm, *bufs):
        sidx = bufs[0:4]
        didx = bufs[4:8]
        rows = bufs[8:12]
        acc = bufs[12]
        gsem = bufs[13:17]
        ssem = bufs[17:21]
        isem = bufs[21:25]

        cid = lax.axis_index("c")
        sid = lax.axis_index("s")
        wid = cid * NSUB + sid

        pltpu.sync_copy(z_hbm.at[pl.ds(sid * RPS, RPS)],
                        acc.at[pl.ds(sid * RPS, RPS)])
        plsc.subcore_barrier()

        base = wid * EPW

        def idx_start(c, p):
            off = base + c * K
            pltpu.async_copy(src_hbm.at[pl.ds(off, K)], sidx[p], isem[p])
            pltpu.async_copy(dst_hbm.at[pl.ds(off, K)], didx[p], isem[p])

        def idx_wait(p):
            pltpu.make_async_copy(src_hbm.at[pl.ds(0, K)], sidx[p],
                                  isem[p]).wait()
            pltpu.make_async_copy(dst_hbm.at[pl.ds(0, K)], didx[p],
                                  isem[p]).wait()

        def gather_start(p):
            pltpu.async_copy(h_hbm.at[sidx[p]], rows[p], gsem[p])

        def gather_wait(p):
            pltpu.make_async_copy(z_hbm.at[pl.ds(0, K)], rows[p],
                                  gsem[p]).wait()

        def scatter_start(p):
            pltpu.async_copy(rows[p], acc.at[didx[p]], ssem[p], add=True)

        def scatter_wait(p):
            pltpu.make_async_copy(rows[p], acc.at[didx[p]], ssem[p]).wait()

        idx_start(0, 0)
        idx_wait(0)
        gather_start(0)
        idx_start(1, 1)

        def body(g, _):
            for u in range(4):
                c = 4 * g + u

                @pl.when(c >= 2)
                def _():
                    scatter_wait((u + 2) % 4)

                @pl.when(c + 1 < NCHUNK)
                def _():
                    idx_wait((u + 1) % 4)
                    gather_start((u + 1) % 4)

                @pl.when(c + 2 < NCHUNK)
                def _():
                    idx_start(c + 2, (u + 2) % 4)

                gather_wait(u)
                scatter_start(u)
            return 0

        lax.fori_loop(0, NCHUNK // 4, body, 0)
        scatter_wait((NCHUNK - 2) % 4)
        scatter_wait((NCHUNK - 1) % 4)
        plsc.subcore_barrier()
        pltpu.sync_copy(acc.at[pl.ds(sid * RPS, RPS)],
                        out_hbm.at[pl.ds(cid * NP + sid * RPS, RPS)])

    return k(h, src, dst, zeros)


def _mlp_layer(x, parts, Wa, ba, Wb, bb):
    def body(x_ref, p0_ref, p1_ref, wa_ref, ba_ref, wb_ref, bb_ref, o_ref):
        m = x_ref[...] + p0_ref[...] + p1_ref[...]
        t = jnp.maximum(
            jnp.dot(m, wa_ref[...], preferred_element_type=jnp.float32)
            + ba_ref[...], 0.0)
        o_ref[...] = jnp.maximum(
            jnp.dot(t, wb_ref[...], preferred_element_type=jnp.float32)
            + bb_ref[...], 0.0)

    return pl.pallas_call(
        body,
        grid=(NBM,),
        in_specs=[
            pl.BlockSpec((BRM, D), lambda i: (i, 0)),
            pl.BlockSpec((BRM, D), lambda i: (i, 0)),
            pl.BlockSpec((BRM, D), lambda i: (i + NBM, 0)),
            pl.BlockSpec((D, D), lambda i: (0, 0)),
            pl.BlockSpec((1, D), lambda i: (0, 0)),
            pl.BlockSpec((D, D), lambda i: (0, 0)),
            pl.BlockSpec((1, D), lambda i: (0, 0)),
        ],
        out_specs=pl.BlockSpec((BRM, D), lambda i: (i, 0)),
        out_shape=jax.ShapeDtypeStruct((NP, D), jnp.float32),
    )(x, parts, parts, Wa, ba.reshape(1, D), Wb, bb.reshape(1, D))


def _head(h1, h2, h3, batch3, Wjk, bjk, Wc1, bc1, g1, bt1, Wc2, bc2):
    inv = float((1.0 + 1e-5) ** -0.5)

    def body(h1_ref, h2_ref, h3_ref, b_ref, wjk_ref, bjk_ref, wc1_ref,
             bc1_ref, g1_ref, bt1_ref, wc2_ref, bc2_ref, o_ref, pooled):
        i = pl.program_id(0)

        @pl.when(i == 0)
        def _():
            pooled[...] = jnp.zeros_like(pooled)

        hcat = jnp.concatenate([h1_ref[...], h2_ref[...], h3_ref[...]],
                               axis=-1)
        xo = jnp.dot(hcat, wjk_ref[...],
                     preferred_element_type=jnp.float32) + bjk_ref[...]
        seg = b_ref[0, 0, :]
        onehot = (jax.lax.broadcasted_iota(jnp.int32, (G, BR), 0)
                  == seg[None, :]).astype(jnp.float32)
        pooled[...] += jnp.dot(onehot, xo, preferred_element_type=jnp.float32)

        @pl.when(i == NB - 1)
        def _():
            c = jnp.dot(pooled[...], wc1_ref[...],
                        preferred_element_type=jnp.float32) + bc1_ref[...]
            c = c * inv * g1_ref[...] + bt1_ref[...]
            c = jnp.maximum(c, 0.0)
            o_ref[...] = jnp.dot(c, wc2_ref[...],
                                 preferred_element_type=jnp.float32) + bc2_ref[...]

    return pl.pallas_call(
        body,
        grid=(NB,),
        in_specs=[
            pl.BlockSpec((BR, D), lambda i: (i, 0)),
            pl.BlockSpec((BR, D), lambda i: (i, 0)),
            pl.BlockSpec((BR, D), lambda i: (i, 0)),
            pl.BlockSpec((1, 1, BR), lambda i: (i, 0, 0)),
            pl.BlockSpec((3 * D, D), lambda i: (0, 0)),
            pl.BlockSpec((1, D), lambda i: (0, 0)),
            pl.BlockSpec((D, 2 * D), lambda i: (0, 0)),
            pl.BlockSpec((1, 2 * D), lambda i: (0, 0)),
            pl.BlockSpec((1, 2 * D), lambda i: (0, 0)),
            pl.BlockSpec((1, 2 * D), lambda i: (0, 0)),
            pl.BlockSpec((2 * D, NC), lambda i: (0, 0)),
            pl.BlockSpec((1, NC), lambda i: (0, 0)),
        ],
        out_specs=pl.BlockSpec((G, NC), lambda i: (0, 0)),
        out_shape=jax.ShapeDtypeStruct((G, NC), jnp.float32),
        scratch_shapes=[pltpu.VMEM((G, D), jnp.float32)],
    )(h1, h2, h3, batch3, Wjk, bjk.reshape(1, D), Wc1, bc1.reshape(1, 2 * D),
      g1.reshape(1, 2 * D), bt1.reshape(1, 2 * D), Wc2, bc2.reshape(1, NC))


def kernel(x, edge_index, batch, W1a, b1a, W1b, b1b, W2a, b2a, W2b, b2b,
           W3a, b3a, W3b, b3b, Wjk, bjk, Wc1, bc1, g1, bt1, Wc2, bc2):
    src = jnp.pad(edge_index[0], (0, EP - E))
    dst = jnp.pad(edge_index[1], (0, EP - E), constant_values=N)
    zeros = jnp.zeros((NP, D), jnp.float32)
    batch3 = batch.reshape(NB, 1, BR)
    xp = jnp.pad(x, ((0, NP - N), (0, 0)))

    p1 = _edge_agg(xp, src, dst, zeros)
    h1 = _mlp_layer(xp, p1, W1a, b1a, W1b, b1b)
    p2 = _edge_agg(h1, src, dst, zeros)
    h2 = _mlp_layer(h1, p2, W2a, b2a, W2b, b2b)
    p3 = _edge_agg(h2, src, dst, zeros)
    h3 = _mlp_layer(h2, p3, W3a, b3a, W3b, b3b)

    return _head(h1, h2, h3, batch3, Wjk, bjk, Wc1, bc1, g1, bt1, Wc2, bc2)


# speedup vs baseline: 2.9102x; 2.7393x over previous
import functools

import jax
import jax.numpy as jnp
from jax import lax
from jax.experimental import pallas as pl
from jax.experimental.pallas import tpu as pltpu
from jax.experimental.pallas import tpu_sc as plsc

N = 10000
E = 640000
D = 128
G = 64
NC = 2

NCORE = 2
NSUB = 16
NW = NCORE * NSUB
EPW = E // NW
K = 80
NCHUNK = EPW // K
NP = 10240
RPS = NP // NSUB

BR = 400
NB = N // BR
BRM = 512
NBM = NP // BRM


def _edge_agg(h, src, dst, zeros):
    mesh = plsc.VectorSubcoreMesh(core_axis_name="c", subcore_axis_name="s")

    @functools.partial(
        pl.kernel,
        out_type=jax.ShapeDtypeStruct((NCORE * NP, D), jnp.float32),
        mesh=mesh,
        scratch_types=[
            pltpu.VMEM((K,), jnp.int32),
            pltpu.VMEM((K,), jnp.int32),
            pltpu.VMEM((K,), jnp.int32),
            pltpu.VMEM((K,), jnp.int32),
            pltpu.VMEM((K, D), jnp.float32),
            pltpu.VMEM((K, D), jnp.float32),
            pltpu.VMEM_SHARED((NP, D), jnp.float32),
            pltpu.SemaphoreType.DMA,
            pltpu.SemaphoreType.DMA,
            pltpu.SemaphoreType.DMA,
            pltpu.SemaphoreType.DMA,
        ],
    )
    def k(h_hbm, src_hbm, dst_hbm, z_hbm, out_hbm,
          sidx0, sidx1, didx0, didx1, rows0, rows1, acc, g0, g1, s0, s1):
        cid = lax.axis_index("c")
        sid = lax.axis_index("s")
        wid = cid * NSUB + sid
        sidx = (sidx0, sidx1)
        didx = (didx0, didx1)
        rows = (rows0, rows1)
        gsem = (g0, g1)
        ssem = (s0, s1)

        pltpu.sync_copy(z_hbm.at[pl.ds(sid * RPS, RPS)],
                        acc.at[pl.ds(sid * RPS, RPS)])
        plsc.subcore_barrier()

        base = wid * EPW

        def idx_load(c, p):
            off = base + c * K
            pltpu.sync_copy(src_hbm.at[pl.ds(off, K)], sidx[p])
            pltpu.sync_copy(dst_hbm.at[pl.ds(off, K)], didx[p])

        def gather_start(p):
            pltpu.async_copy(h_hbm.at[sidx[p]], rows[p], gsem[p])

        def gather_wait(p):
            pltpu.make_async_copy(z_hbm.at[pl.ds(0, K)], rows[p],
                                  gsem[p]).wait()

        def scatter_start(p):
            pltpu.async_copy(rows[p], acc.at[didx[p]], ssem[p], add=True)

        def scatter_wait(p):
            pltpu.make_async_copy(rows[p], acc.at[didx[p]], ssem[p]).wait()

        idx_load(0, 0)
        gather_start(0)

        def body(c2, _):
            for u in range(2):
                c = 2 * c2 + u
                q = 1 - u

                @pl.when(c + 1 < NCHUNK)
                def _():
                    @pl.when(c >= 1)
                    def _():
                        scatter_wait(q)
                    idx_load(c + 1, q)
                    gather_start(q)

                gather_wait(u)
                scatter_start(u)
            return 0

        lax.fori_loop(0, NCHUNK // 2, body, 0)
        scatter_wait(0)
        scatter_wait(1)
        plsc.subcore_barrier()
        pltpu.sync_copy(acc.at[pl.ds(sid * RPS, RPS)],
                        out_hbm.at[pl.ds(cid * NP + sid * RPS, RPS)])

    return k(h, src, dst, zeros)


def _mlp_layer(x, parts, Wa, ba, Wb, bb):
    def body(x_ref, p0_ref, p1_ref, wa_ref, ba_ref, wb_ref, bb_ref, o_ref):
        m = x_ref[...] + p0_ref[...] + p1_ref[...]
        t = jnp.maximum(
            jnp.dot(m, wa_ref[...], preferred_element_type=jnp.float32)
            + ba_ref[...], 0.0)
        o_ref[...] = jnp.maximum(
            jnp.dot(t, wb_ref[...], preferred_element_type=jnp.float32)
            + bb_ref[...], 0.0)

    return pl.pallas_call(
        body,
        grid=(NBM,),
        in_specs=[
            pl.BlockSpec((BRM, D), lambda i: (i, 0)),
            pl.BlockSpec((BRM, D), lambda i: (i, 0)),
            pl.BlockSpec((BRM, D), lambda i: (i + NBM, 0)),
            pl.BlockSpec((D, D), lambda i: (0, 0)),
            pl.BlockSpec((1, D), lambda i: (0, 0)),
            pl.BlockSpec((D, D), lambda i: (0, 0)),
            pl.BlockSpec((1, D), lambda i: (0, 0)),
        ],
        out_specs=pl.BlockSpec((BRM, D), lambda i: (i, 0)),
        out_shape=jax.ShapeDtypeStruct((NP, D), jnp.float32),
    )(x, parts, parts, Wa, ba.reshape(1, D), Wb, bb.reshape(1, D))


def _head(h1, h2, h3, batch3, Wjk, bjk, Wc1, bc1, g1, bt1, Wc2, bc2):
    inv = float((1.0 + 1e-5) ** -0.5)

    def body(h1_ref, h2_ref, h3_ref, b_ref, wjk_ref, bjk_ref, wc1_ref,
             bc1_ref, g1_ref, bt1_ref, wc2_ref, bc2_ref, o_ref, pooled):
        i = pl.program_id(0)

        @pl.when(i == 0)
        def _():
            pooled[...] = jnp.zeros_like(pooled)

        hcat = jnp.concatenate([h1_ref[...], h2_ref[...], h3_ref[...]],
                               axis=-1)
        xo = jnp.dot(hcat, wjk_ref[...],
                     preferred_element_type=jnp.float32) + bjk_ref[...]
        seg = b_ref[0, 0, :]
        onehot = (jax.lax.broadcasted_iota(jnp.int32, (G, BR), 0)
                  == seg[None, :]).astype(jnp.float32)
        pooled[...] += jnp.dot(onehot, xo, preferred_element_type=jnp.float32)

        @pl.when(i == NB - 1)
        def _():
            c = jnp.dot(pooled[...], wc1_ref[...],
                        preferred_element_type=jnp.float32) + bc1_ref[...]
            c = c * inv * g1_ref[...] + bt1_ref[...]
            c = jnp.maximum(c, 0.0)
            o_ref[...] = jnp.dot(c, wc2_ref[...],
                                 preferred_element_type=jnp.float32) + bc2_ref[...]

    return pl.pallas_call(
        body,
        grid=(NB,),
        in_specs=[
            pl.BlockSpec((BR, D), lambda i: (i, 0)),
            pl.BlockSpec((BR, D), lambda i: (i, 0)),
            pl.BlockSpec((BR, D), lambda i: (i, 0)),
            pl.BlockSpec((1, 1, BR), lambda i: (i, 0, 0)),
            pl.BlockSpec((3 * D, D), lambda i: (0, 0)),
            pl.BlockSpec((1, D), lambda i: (0, 0)),
            pl.BlockSpec((D, 2 * D), lambda i: (0, 0)),
            pl.BlockSpec((1, 2 * D), lambda i: (0, 0)),
            pl.BlockSpec((1, 2 * D), lambda i: (0, 0)),
            pl.BlockSpec((1, 2 * D), lambda i: (0, 0)),
            pl.BlockSpec((2 * D, NC), lambda i: (0, 0)),
            pl.BlockSpec((1, NC), lambda i: (0, 0)),
        ],
        out_specs=pl.BlockSpec((G, NC), lambda i: (0, 0)),
        out_shape=jax.ShapeDtypeStruct((G, NC), jnp.float32),
        scratch_shapes=[pltpu.VMEM((G, D), jnp.float32)],
    )(h1, h2, h3, batch3, Wjk, bjk.reshape(1, D), Wc1, bc1.reshape(1, 2 * D),
      g1.reshape(1, 2 * D), bt1.reshape(1, 2 * D), Wc2, bc2.reshape(1, NC))


def kernel(x, edge_index, batch, W1a, b1a, W1b, b1b, W2a, b2a, W2b, b2b,
           W3a, b3a, W3b, b3b, Wjk, bjk, Wc1, bc1, g1, bt1, Wc2, bc2):
    src = edge_index[0]
    dst = edge_index[1]
    zeros = jnp.zeros((NP, D), jnp.float32)
    batch3 = batch.reshape(NB, 1, BR)
    xp = jnp.pad(x, ((0, NP - N), (0, 0)))

    p1 = _edge_agg(xp, src, dst, zeros)
    h1 = _mlp_layer(xp, p1, W1a, b1a, W1b, b1b)
    p2 = _edge_agg(h1, src, dst, zeros)
    h2 = _mlp_layer(h1, p2, W2a, b2a, W2b, b2b)
    p3 = _edge_agg(h2, src, dst, zeros)
    h3 = _mlp_layer(h2, p3, W3a, b3a, W3b, b3b)

    return _head(h1, h2, h3, batch3, Wjk, bjk, Wc1, bc1, g1, bt1, Wc2, bc2)


# speedup vs baseline: 3.6755x; 1.2630x over previous
import functools

import jax
import jax.numpy as jnp
from jax import lax
from jax.experimental import pallas as pl
from jax.experimental.pallas import tpu as pltpu
from jax.experimental.pallas import tpu_sc as plsc

N = 10000
E = 640000
D = 128
G = 64
NC = 2

NCORE = 2
NSUB = 16
NW = NCORE * NSUB
EPW = E // NW
K = 80
NCHUNK = EPW // K
NP = 10240
RPS = NP // NSUB

BR = 400
NB = N // BR
BRM = 512
NBM = NP // BRM


def _edge_agg(h, src, dst, zeros):
    mesh = plsc.VectorSubcoreMesh(core_axis_name="c", subcore_axis_name="s")

    @functools.partial(
        pl.kernel,
        out_type=jax.ShapeDtypeStruct((NCORE * NP, D), jnp.float32),
        mesh=mesh,
        scratch_types=[
            pltpu.VMEM((K,), jnp.int32),
            pltpu.VMEM((K,), jnp.int32),
            pltpu.VMEM((K,), jnp.int32),
            pltpu.VMEM((K,), jnp.int32),
            pltpu.VMEM((K, D), jnp.float32),
            pltpu.VMEM((K, D), jnp.float32),
            pltpu.VMEM_SHARED((NP, D), jnp.float32),
            pltpu.SemaphoreType.DMA,
            pltpu.SemaphoreType.DMA,
            pltpu.SemaphoreType.DMA,
            pltpu.SemaphoreType.DMA,
            pltpu.SemaphoreType.DMA,
            pltpu.SemaphoreType.DMA,
        ],
    )
    def k(h_hbm, src_hbm, dst_hbm, z_hbm, out_hbm,
          sidx0, sidx1, didx0, didx1, rows0, rows1, acc,
          g0, g1, s0, s1, i0, i1):
        cid = lax.axis_index("c")
        sid = lax.axis_index("s")
        wid = cid * NSUB + sid
        sidx = (sidx0, sidx1)
        didx = (didx0, didx1)
        rows = (rows0, rows1)
        gsem = (g0, g1)
        ssem = (s0, s1)
        isem = (i0, i1)

        pltpu.sync_copy(z_hbm.at[pl.ds(sid * RPS, RPS)],
                        acc.at[pl.ds(sid * RPS, RPS)])
        plsc.subcore_barrier()

        base = wid * EPW

        def src_idx_start(c, p):
            pltpu.async_copy(src_hbm.at[pl.ds(base + c * K, K)], sidx[p],
                             isem[p])

        def dst_idx_start(c, p):
            pltpu.async_copy(dst_hbm.at[pl.ds(base + c * K, K)], didx[p],
                             isem[p])

        def idx_wait(p):
            pltpu.make_async_copy(src_hbm.at[pl.ds(0, K)], sidx[p],
                                  isem[p]).wait()
            pltpu.make_async_copy(dst_hbm.at[pl.ds(0, K)], didx[p],
                                  isem[p]).wait()

        def gather_start(p):
            pltpu.async_copy(h_hbm.at[sidx[p]], rows[p], gsem[p])

        def gather_wait(p):
            pltpu.make_async_copy(z_hbm.at[pl.ds(0, K)], rows[p],
                                  gsem[p]).wait()

        def scatter_start(p):
            pltpu.async_copy(rows[p], acc.at[didx[p]], ssem[p], add=True)

        def scatter_wait(p):
            pltpu.make_async_copy(rows[p], acc.at[didx[p]], ssem[p]).wait()

        src_idx_start(0, 0)
        dst_idx_start(0, 0)
        idx_wait(0)
        gather_start(0)

        def body(c2, _):
            for u in range(2):
                c = 2 * c2 + u
                q = 1 - u

                @pl.when(c + 1 < NCHUNK)
                def _():
                    src_idx_start(c + 1, q)
                    @pl.when(c >= 1)
                    def _():
                        scatter_wait(q)
                    dst_idx_start(c + 1, q)
                    idx_wait(q)
                    gather_start(q)

                gather_wait(u)
                scatter_start(u)
            return 0

        lax.fori_loop(0, NCHUNK // 2, body, 0)
        scatter_wait(0)
        scatter_wait(1)
        plsc.subcore_barrier()
        pltpu.sync_copy(acc.at[pl.ds(sid * RPS, RPS)],
                        out_hbm.at[pl.ds(cid * NP + sid * RPS, RPS)])

    return k(h, src, dst, zeros)


def _mlp_layer(x, parts, Wa, ba, Wb, bb):
    def body(x_ref, p0_ref, p1_ref, wa_ref, ba_ref, wb_ref, bb_ref, o_ref):
        m = x_ref[...] + p0_ref[...] + p1_ref[...]
        t = jnp.maximum(
            jnp.dot(m, wa_ref[...], preferred_element_type=jnp.float32)
            + ba_ref[...], 0.0)
        o_ref[...] = jnp.maximum(
            jnp.dot(t, wb_ref[...], preferred_element_type=jnp.float32)
            + bb_ref[...], 0.0)

    return pl.pallas_call(
        body,
        grid=(NBM,),
        in_specs=[
            pl.BlockSpec((BRM, D), lambda i: (i, 0)),
            pl.BlockSpec((BRM, D), lambda i: (i, 0)),
            pl.BlockSpec((BRM, D), lambda i: (i + NBM, 0)),
            pl.BlockSpec((D, D), lambda i: (0, 0)),
            pl.BlockSpec((1, D), lambda i: (0, 0)),
            pl.BlockSpec((D, D), lambda i: (0, 0)),
            pl.BlockSpec((1, D), lambda i: (0, 0)),
        ],
        out_specs=pl.BlockSpec((BRM, D), lambda i: (i, 0)),
        out_shape=jax.ShapeDtypeStruct((NP, D), jnp.float32),
    )(x, parts, parts, Wa, ba.reshape(1, D), Wb, bb.reshape(1, D))


def _head(h1, h2, h3, batch3, Wjk, bjk, Wc1, bc1, g1, bt1, Wc2, bc2):
    inv = float((1.0 + 1e-5) ** -0.5)

    def body(h1_ref, h2_ref, h3_ref, b_ref, wjk_ref, bjk_ref, wc1_ref,
             bc1_ref, g1_ref, bt1_ref, wc2_ref, bc2_ref, o_ref, pooled):
        i = pl.program_id(0)

        @pl.when(i == 0)
        def _():
            pooled[...] = jnp.zeros_like(pooled)

        hcat = jnp.concatenate([h1_ref[...], h2_ref[...], h3_ref[...]],
                               axis=-1)
        xo = jnp.dot(hcat, wjk_ref[...],
                     preferred_element_type=jnp.float32) + bjk_ref[...]
        seg = b_ref[0, 0, :]
        onehot = (jax.lax.broadcasted_iota(jnp.int32, (G, BR), 0)
                  == seg[None, :]).astype(jnp.float32)
        pooled[...] += jnp.dot(onehot, xo, preferred_element_type=jnp.float32)

        @pl.when(i == NB - 1)
        def _():
            c = jnp.dot(pooled[...], wc1_ref[...],
                        preferred_element_type=jnp.float32) + bc1_ref[...]
            c = c * inv * g1_ref[...] + bt1_ref[...]
            c = jnp.maximum(c, 0.0)
            o_ref[...] = jnp.dot(c, wc2_ref[...],
                                 preferred_element_type=jnp.float32) + bc2_ref[...]

    return pl.pallas_call(
        body,
        grid=(NB,),
        in_specs=[
            pl.BlockSpec((BR, D), lambda i: (i, 0)),
            pl.BlockSpec((BR, D), lambda i: (i, 0)),
            pl.BlockSpec((BR, D), lambda i: (i, 0)),
            pl.BlockSpec((1, 1, BR), lambda i: (i, 0, 0)),
            pl.BlockSpec((3 * D, D), lambda i: (0, 0)),
            pl.BlockSpec((1, D), lambda i: (0, 0)),
            pl.BlockSpec((D, 2 * D), lambda i: (0, 0)),
            pl.BlockSpec((1, 2 * D), lambda i: (0, 0)),
            pl.BlockSpec((1, 2 * D), lambda i: (0, 0)),
            pl.BlockSpec((1, 2 * D), lambda i: (0, 0)),
            pl.BlockSpec((2 * D, NC), lambda i: (0, 0)),
            pl.BlockSpec((1, NC), lambda i: (0, 0)),
        ],
        out_specs=pl.BlockSpec((G, NC), lambda i: (0, 0)),
        out_shape=jax.ShapeDtypeStruct((G, NC), jnp.float32),
        scratch_shapes=[pltpu.VMEM((G, D), jnp.float32)],
    )(h1, h2, h3, batch3, Wjk, bjk.reshape(1, D), Wc1, bc1.reshape(1, 2 * D),
      g1.reshape(1, 2 * D), bt1.reshape(1, 2 * D), Wc2, bc2.reshape(1, NC))


def kernel(x, edge_index, batch, W1a, b1a, W1b, b1b, W2a, b2a, W2b, b2b,
           W3a, b3a, W3b, b3b, Wjk, bjk, Wc1, bc1, g1, bt1, Wc2, bc2):
    src = edge_index[0]
    dst = edge_index[1]
    zeros = jnp.zeros((NP, D), jnp.float32)
    batch3 = batch.reshape(NB, 1, BR)
    xp = jnp.pad(x, ((0, NP - N), (0, 0)))

    p1 = _edge_agg(xp, src, dst, zeros)
    h1 = _mlp_layer(xp, p1, W1a, b1a, W1b, b1b)
    p2 = _edge_agg(h1, src, dst, zeros)
    h2 = _mlp_layer(h1, p2, W2a, b2a, W2b, b2b)
    p3 = _edge_agg(h2, src, dst, zeros)
    h3 = _mlp_layer(h2, p3, W3a, b3a, W3b, b3b)

    return _head(h1, h2, h3, batch3, Wjk, bjk, Wc1, bc1, g1, bt1, Wc2, bc2)
